# baseline (speedup 1.0000x reference)
import functools

import jax
import jax.numpy as jnp
from jax import lax
from jax.experimental import pallas as pl
from jax.experimental.pallas import tpu as pltpu
from jax.experimental.pallas import tpu_sc as plsc

NUM_ROWS = 16384
NUM_COLS = 26
DIM = 64

NC = 2
NS = 16
NW = NC * NS
B = NUM_ROWS * NUM_COLS
B_PER_W = B // NW
CHUNK = 128
NCH = B_PER_W // CHUNK


@functools.partial(
    pl.kernel,
    mesh=plsc.VectorSubcoreMesh(core_axis_name="c", subcore_axis_name="s"),
    out_type=jax.ShapeDtypeStruct((NW, NCH, CHUNK, DIM), jnp.float32),
    scratch_types=[
        pltpu.VMEM((NCH, CHUNK), jnp.int32),
        pltpu.VMEM((CHUNK, DIM), jnp.float32),
        pltpu.SemaphoreType.DMA,
    ],
    compiler_params=pltpu.CompilerParams(use_tc_tiling_on_sc=False),
)
def _gather_kernel(idx_hbm, table_hbm, out_hbm, idx_v, rows_v, sem):
    wid = lax.axis_index("s") * NC + lax.axis_index("c")
    pltpu.sync_copy(idx_hbm.at[wid], idx_v)

    def body(g, carry):
        pltpu.async_copy(table_hbm.at[idx_v.at[g]], rows_v, sem).wait()
        pltpu.sync_copy(rows_v, out_hbm.at[wid, g])
        return carry

    lax.fori_loop(0, NCH, body, 0)


def kernel(x, W):
    idx = x.reshape(NW, NCH, CHUNK).astype(jnp.int32)
    out = _gather_kernel(idx, W)
    return out.reshape(NUM_ROWS, NUM_COLS, DIM)


# speedup vs baseline: 1.0795x; 1.0795x over previous
import functools

import jax
import jax.numpy as jnp
from jax import lax
from jax.experimental import pallas as pl
from jax.experimental.pallas import tpu as pltpu
from jax.experimental.pallas import tpu_sc as plsc

NUM_ROWS = 16384
NUM_COLS = 26
DIM = 64

NC = 2
NS = 16
NW = NC * NS
B = NUM_ROWS * NUM_COLS
B_PER_W = B // NW
CHUNK = 128
K = 2
SLOT = K * CHUNK
NSLOT = B_PER_W // SLOT
NBUF = 4
PD = NBUF - 2


@functools.partial(
    pl.kernel,
    mesh=plsc.VectorSubcoreMesh(core_axis_name="c", subcore_axis_name="s"),
    out_type=jax.ShapeDtypeStruct((NW, NSLOT, SLOT, DIM), jnp.float32),
    scratch_types=[
        pltpu.VMEM((NSLOT * K, CHUNK), jnp.int32),
        pltpu.VMEM((NBUF, SLOT, DIM), jnp.float32),
        pltpu.SemaphoreType.DMA((NBUF,)),
        pltpu.SemaphoreType.DMA((NBUF,)),
    ],
    compiler_params=pltpu.CompilerParams(use_tc_tiling_on_sc=False),
)
def _gather_kernel(idx_hbm, table_hbm, out_hbm, idx_v, rows_v, gsem, osem):
    wid = lax.axis_index("s") * NC + lax.axis_index("c")
    pltpu.sync_copy(idx_hbm.at[wid], idx_v)

    def start_gather(slot, buf):
        for j in range(K):
            pltpu.async_copy(
                table_hbm.at[idx_v.at[slot * K + j]],
                rows_v.at[buf, pl.ds(j * CHUNK, CHUNK)],
                gsem.at[buf],
            )

    for s in range(PD):
        start_gather(s, s)

    def body(s, carry):
        b = lax.rem(s, NBUF)
        p = s + PD
        bp = lax.rem(p, NBUF)

        @pl.when(p < NSLOT)
        def _prefetch():
            @pl.when(s >= NBUF - PD)
            def _():
                pltpu.make_async_copy(
                    rows_v.at[bp], out_hbm.at[wid, 0], osem.at[bp]
                ).wait()

            start_gather(p, bp)

        for j in range(K):
            pltpu.make_async_copy(
                table_hbm.at[idx_v.at[j]],
                rows_v.at[b, pl.ds(j * CHUNK, CHUNK)],
                gsem.at[b],
            ).wait()
        pltpu.async_copy(rows_v.at[b], out_hbm.at[wid, s], osem.at[b])
        return carry

    lax.fori_loop(0, NSLOT, body, 0)

    for b in range(NBUF):
        pltpu.make_async_copy(
            rows_v.at[b], out_hbm.at[wid, 0], osem.at[b]
        ).wait()


def kernel(x, W):
    idx = x.reshape(NW, NSLOT * K, CHUNK).astype(jnp.int32)
    out = _gather_kernel(idx, W)
    return out.reshape(NUM_ROWS, NUM_COLS, DIM)
